# baseline (speedup 1.0000x reference)
import functools

import jax
import jax.numpy as jnp
from jax import lax
from jax.experimental import pallas as pl
from jax.experimental.pallas import tpu as pltpu
from jax.experimental.pallas import tpu_sc as plsc

_N = 10000
_E = 320000
_D = 128
_D2 = 16
_NP = 10240
_NC, _NS = 2, 16
_NW = _NC * _NS
_CH = 80
_NCH = 128
_EPT = _NCH * _CH
_EPAD = _NW * _EPT
_EPS = _E // _NS

_mesh = plsc.VectorSubcoreMesh(core_axis_name="c", subcore_axis_name="s")


def _deg_body(dst_hbm, deg_out, dst_v, hist_v, blk_v, acc_v, spm):
    cid = lax.axis_index("c")
    sid = lax.axis_index("s")
    zeros16 = jnp.zeros((16,), jnp.float32)
    ones16 = jnp.ones((16,), jnp.float32)

    @pl.when(cid == 0)
    def _():
        def zero(i, _):
            hist_v[pl.ds(i * 16, 16)] = zeros16
            return 0
        lax.fori_loop(0, _NP // 16, zero, 0)

        pltpu.sync_copy(dst_hbm.at[pl.ds(sid * _EPS, _EPS)], dst_v)

        def count(i, _):
            idx = dst_v[pl.ds(i * 16, 16)]
            plsc.addupdate_scatter(hist_v, [idx], ones16)
            return 0
        lax.fori_loop(0, _EPS // 16, count, 0)

        pltpu.sync_copy(hist_v, spm.at[sid])

    plsc.subcore_barrier()

    @pl.when(cid == 0)
    def _():
        pltpu.sync_copy(spm.at[:, pl.ds(sid * 640, 640)], blk_v)

        def reduce(j, _):
            s = blk_v[0, pl.ds(j * 16, 16)]
            for r in range(1, _NS):
                s = s + blk_v[r, pl.ds(j * 16, 16)]
            acc_v[pl.ds(j * 16, 16)] = s
            return 0
        lax.fori_loop(0, 640 // 16, reduce, 0)

        pltpu.sync_copy(acc_v, deg_out.at[pl.ds(sid * 640, 640)])


_sc_params = pltpu.CompilerParams(needs_layout_passes=False,
                                  use_tc_tiling_on_sc=False)

_deg_call = functools.partial(
    pl.kernel,
    out_type=jax.ShapeDtypeStruct((_NP,), jnp.float32),
    mesh=_mesh,
    compiler_params=_sc_params,
    scratch_types=[
        pltpu.VMEM((_EPS,), jnp.int32),
        pltpu.VMEM((_NP,), jnp.float32),
        pltpu.VMEM((_NS, 640), jnp.float32),
        pltpu.VMEM((640,), jnp.float32),
        pltpu.VMEM_SHARED((_NS, _NP), jnp.float32),
    ],
)(_deg_body)


_GRP = 2
_NBUF = 2 * _GRP
_NIDX = 4


def _agg_body(y_hbm, src_hbm, dst_hbm, zero_hbm, out_hbm,
              idxs_v, idxd_v, rows_v, spm, isems, isemd, gsem, ssem):
    cid = lax.axis_index("c")
    sid = lax.axis_index("s")
    wid = sid * _NC + cid
    ng = _NCH // _GRP

    pltpu.sync_copy(zero_hbm.at[pl.ds(sid * 640, 640)],
                    spm.at[pl.ds(sid * 640, 640)])
    pltpu.async_copy(src_hbm.at[wid, pl.ds(0, _GRP)], idxs_v.at[0], isems.at[0])
    pltpu.async_copy(dst_hbm.at[wid, pl.ds(0, _GRP)], idxd_v.at[0], isemd.at[0])
    plsc.subcore_barrier()

    drow = y_hbm.at[pl.ds(0, _CH)]
    didx = src_hbm.at[0, pl.ds(0, _GRP)]

    def group(g, _):
        p = (g % 2) * _GRP
        q = g % _NIDX

        @pl.when(g >= 2)
        def _():
            for b in range(_GRP):
                pltpu.make_async_copy(drow, rows_v.at[p + b],
                                      ssem.at[p + b]).wait()

        @pl.when(g + 1 < ng)
        def _():
            qn = (g + 1) % _NIDX
            pltpu.async_copy(src_hbm.at[wid, pl.ds((g + 1) * _GRP, _GRP)],
                             idxs_v.at[qn], isems.at[qn])
            pltpu.async_copy(dst_hbm.at[wid, pl.ds((g + 1) * _GRP, _GRP)],
                             idxd_v.at[qn], isemd.at[qn])

        pltpu.make_async_copy(didx, idxs_v.at[q], isems.at[q]).wait()
        pltpu.make_async_copy(didx, idxd_v.at[q], isemd.at[q]).wait()

        gathers = []
        for b in range(_GRP):
            gathers.append(pltpu.async_copy(
                y_hbm.at[idxs_v.at[q, b]], rows_v.at[p + b], gsem.at[p + b]))
        for b in range(_GRP):
            gathers[b].wait()
            pltpu.async_copy(rows_v.at[p + b], spm.at[idxd_v.at[q, b]],
                             ssem.at[p + b], add=True)
        return 0

    lax.fori_loop(0, ng, group, 0)
    for b in range(_NBUF):
        pltpu.make_async_copy(drow, rows_v.at[b], ssem.at[b]).wait()

    plsc.subcore_barrier()
    pltpu.sync_copy(spm.at[pl.ds(sid * 640, 640)],
                    out_hbm.at[cid, pl.ds(sid * 640, 640)])


def _make_agg(width):
    return functools.partial(
        pl.kernel,
        out_type=jax.ShapeDtypeStruct((_NC, _NP, width), jnp.float32),
        mesh=_mesh,
        compiler_params=_sc_params,
        scratch_types=[
            pltpu.VMEM((_NIDX, _GRP, _CH), jnp.int32),
            pltpu.VMEM((_NIDX, _GRP, _CH), jnp.int32),
            pltpu.VMEM((_NBUF, _CH, width), jnp.float32),
            pltpu.VMEM_SHARED((_NP, width), jnp.float32),
            pltpu.SemaphoreType.DMA((_NIDX,)),
            pltpu.SemaphoreType.DMA((_NIDX,)),
            pltpu.SemaphoreType.DMA((_NBUF,)),
            pltpu.SemaphoreType.DMA((_NBUF,)),
        ],
    )(_agg_body)


_agg_call_d = _make_agg(_D)
_agg_call_2 = _make_agg(_D2)


_BR = 1000


def _y1_body(deg_ref, x_ref, w_ref, o_ref):
    dinv = lax.rsqrt(deg_ref[...] + 1.0)
    o_ref[...] = dinv * jnp.dot(x_ref[...], w_ref[...],
                                preferred_element_type=jnp.float32)


def _y1_call(degc, x, w1):
    return pl.pallas_call(
        _y1_body,
        grid=(_N // _BR,),
        in_specs=[
            pl.BlockSpec((_BR, 1), lambda i: (i, 0)),
            pl.BlockSpec((_BR, _D), lambda i: (i, 0)),
            pl.BlockSpec((_D, _D), lambda i: (0, 0)),
        ],
        out_specs=pl.BlockSpec((_BR, _D), lambda i: (i, 0)),
        out_shape=jax.ShapeDtypeStruct((_N, _D), jnp.float32),
    )(degc, x, w1)


def _mid_body(deg_ref, agg_ref, y1_ref, b1_ref, w2_ref, o_ref):
    dinv = lax.rsqrt(deg_ref[...] + 1.0)
    h = dinv * (agg_ref[0] + agg_ref[1] + y1_ref[...]) + b1_ref[...]
    h = jnp.maximum(h, 0.0)
    y2 = dinv * jnp.dot(h, w2_ref[...], preferred_element_type=jnp.float32)
    o_ref[...] = y2[:, :_D2]


def _mid_call(degc, agg1, y1, b1r, w2p):
    return pl.pallas_call(
        _mid_body,
        grid=(_N // _BR,),
        in_specs=[
            pl.BlockSpec((_BR, 1), lambda i: (i, 0)),
            pl.BlockSpec((_NC, _BR, _D), lambda i: (0, i, 0)),
            pl.BlockSpec((_BR, _D), lambda i: (i, 0)),
            pl.BlockSpec((1, _D), lambda i: (0, 0)),
            pl.BlockSpec((_D, _D), lambda i: (0, 0)),
        ],
        out_specs=pl.BlockSpec((_BR, _D2), lambda i: (i, 0)),
        out_shape=jax.ShapeDtypeStruct((_N, _D2), jnp.float32),
    )(degc, agg1, y1, b1r, w2p)


def _fin_body(deg_ref, agg_ref, y2_ref, b2_ref, o_ref):
    dinv = lax.rsqrt(deg_ref[...] + 1.0)
    o_ref[...] = dinv * (agg_ref[0] + agg_ref[1] + y2_ref[...]) + b2_ref[...]


def _fin_call(degc, agg2, y2, b2r):
    return pl.pallas_call(
        _fin_body,
        grid=(_N // _BR,),
        in_specs=[
            pl.BlockSpec((_BR, 1), lambda i: (i, 0)),
            pl.BlockSpec((_NC, _BR, _D2), lambda i: (0, i, 0)),
            pl.BlockSpec((_BR, _D2), lambda i: (i, 0)),
            pl.BlockSpec((1, _D2), lambda i: (0, 0)),
        ],
        out_specs=pl.BlockSpec((_BR, _D2), lambda i: (i, 0)),
        out_shape=jax.ShapeDtypeStruct((_N, _D2), jnp.float32),
    )(degc, agg2, y2, b2r)


def kernel(x, edge_index, W1, b1, W2, b2):
    x = x.astype(jnp.float32)
    npad = _EPAD - _E
    src3 = jnp.concatenate(
        [edge_index[0], jnp.zeros((npad,), jnp.int32)]).reshape(_NW, _NCH, _CH)
    dst3 = jnp.concatenate(
        [edge_index[1], jnp.full((npad,), _N, jnp.int32)]).reshape(_NW, _NCH, _CH)

    deg = _deg_call(edge_index[1])
    degc = deg[:_N].reshape(_N, 1)

    y1 = _y1_call(degc, x, W1)
    zeros1 = jnp.zeros((_NP, _D), jnp.float32)
    agg1 = _agg_call_d(y1, src3, dst3, zeros1)

    w2p = jnp.pad(W2, ((0, 0), (0, _D - W2.shape[1])))
    y2 = _mid_call(degc, agg1, y1, b1.reshape(1, _D), w2p)

    zeros2 = jnp.zeros((_NP, _D2), jnp.float32)
    agg2 = _agg_call_2(y2, src3, dst3, zeros2)

    b2p = jnp.pad(b2, (0, _D2 - b2.shape[0])).reshape(1, _D2)
    out16 = _fin_call(degc, agg2, y2, b2p)
    return out16[:, :8]


# speedup vs baseline: 1.7856x; 1.7856x over previous
import functools

import jax
import jax.numpy as jnp
from jax import lax
from jax.experimental import pallas as pl
from jax.experimental.pallas import tpu as pltpu
from jax.experimental.pallas import tpu_sc as plsc

_N = 10000
_E = 320000
_D = 128
_D2 = 16
_NP = 10240
_NC, _NS = 2, 16
_NW = _NC * _NS
_CH = 112
_NCH = 90
_EPT = _NCH * _CH
_EPAD = _NW * _EPT
_EPS = _E // _NS

_mesh = plsc.VectorSubcoreMesh(core_axis_name="c", subcore_axis_name="s")


def _deg_body(dst_hbm, deg_out, dst_v, hist_v, blk_v, acc_v, spm):
    cid = lax.axis_index("c")
    sid = lax.axis_index("s")
    zeros16 = jnp.zeros((16,), jnp.float32)
    ones16 = jnp.ones((16,), jnp.float32)

    @pl.when(cid == 0)
    def _():
        def zero(i, _):
            hist_v[pl.ds(i * 16, 16)] = zeros16
            return 0
        lax.fori_loop(0, _NP // 16, zero, 0)

        pltpu.sync_copy(dst_hbm.at[pl.ds(sid * _EPS, _EPS)], dst_v)

        def count(i, _):
            idx = dst_v[pl.ds(i * 16, 16)]
            plsc.addupdate_scatter(hist_v, [idx], ones16)
            return 0
        lax.fori_loop(0, _EPS // 16, count, 0)

        pltpu.sync_copy(hist_v, spm.at[sid])

    plsc.subcore_barrier()

    @pl.when(cid == 0)
    def _():
        pltpu.sync_copy(spm.at[:, pl.ds(sid * 640, 640)], blk_v)

        def reduce(j, _):
            s = blk_v[0, pl.ds(j * 16, 16)]
            for r in range(1, _NS):
                s = s + blk_v[r, pl.ds(j * 16, 16)]
            acc_v[pl.ds(j * 16, 16)] = s
            return 0
        lax.fori_loop(0, 640 // 16, reduce, 0)

        pltpu.sync_copy(acc_v, deg_out.at[pl.ds(sid * 640, 640)])


_sc_params = pltpu.CompilerParams(needs_layout_passes=False,
                                  use_tc_tiling_on_sc=False)

_deg_call = functools.partial(
    pl.kernel,
    out_type=jax.ShapeDtypeStruct((_NP,), jnp.float32),
    mesh=_mesh,
    compiler_params=_sc_params,
    scratch_types=[
        pltpu.VMEM((_EPS,), jnp.int32),
        pltpu.VMEM((_NP,), jnp.float32),
        pltpu.VMEM((_NS, 640), jnp.float32),
        pltpu.VMEM((640,), jnp.float32),
        pltpu.VMEM_SHARED((_NS, _NP), jnp.float32),
    ],
)(_deg_body)


def _agg_body(y_hbm, src_hbm, dst_hbm, zero_hbm, out_hbm,
              src_v, dst_v, rows_v, spm, gsem, ssem):
    cid = lax.axis_index("c")
    sid = lax.axis_index("s")
    wid = sid * _NC + cid

    pltpu.sync_copy(zero_hbm.at[pl.ds(sid * 640, 640)],
                    spm.at[pl.ds(sid * 640, 640)])
    pltpu.sync_copy(src_hbm.at[wid], src_v)
    pltpu.sync_copy(dst_hbm.at[wid], dst_v)
    plsc.subcore_barrier()

    drow = y_hbm.at[pl.ds(0, _CH)]

    def chunk(j, _):
        p = j % 2

        @pl.when(j >= 2)
        def _():
            pltpu.make_async_copy(drow, rows_v.at[p], ssem.at[p]).wait()

        pltpu.async_copy(y_hbm.at[src_v.at[j]], rows_v.at[p],
                         gsem.at[p]).wait()
        pltpu.async_copy(rows_v.at[p], spm.at[dst_v.at[j]],
                         ssem.at[p], add=True)
        return 0

    lax.fori_loop(0, _NCH, chunk, 0)
    for b in range(2):
        pltpu.make_async_copy(drow, rows_v.at[b], ssem.at[b]).wait()

    plsc.subcore_barrier()
    pltpu.sync_copy(spm.at[pl.ds(sid * 640, 640)],
                    out_hbm.at[cid, pl.ds(sid * 640, 640)])


def _make_agg(width):
    return functools.partial(
        pl.kernel,
        out_type=jax.ShapeDtypeStruct((_NC, _NP, width), jnp.float32),
        mesh=_mesh,
        compiler_params=_sc_params,
        scratch_types=[
            pltpu.VMEM((_NCH, _CH), jnp.int32),
            pltpu.VMEM((_NCH, _CH), jnp.int32),
            pltpu.VMEM((2, _CH, width), jnp.float32),
            pltpu.VMEM_SHARED((_NP, width), jnp.float32),
            pltpu.SemaphoreType.DMA((2,)),
            pltpu.SemaphoreType.DMA((2,)),
        ],
    )(_agg_body)


_agg_call_d = _make_agg(_D)
_agg_call_2 = _make_agg(_D2)


_BR = 1000


def _y1_body(deg_ref, x_ref, w_ref, o_ref):
    dinv = lax.rsqrt(deg_ref[...] + 1.0)
    o_ref[...] = dinv * jnp.dot(x_ref[...], w_ref[...],
                                preferred_element_type=jnp.float32)


def _y1_call(degc, x, w1):
    return pl.pallas_call(
        _y1_body,
        grid=(_N // _BR,),
        in_specs=[
            pl.BlockSpec((_BR, 1), lambda i: (i, 0)),
            pl.BlockSpec((_BR, _D), lambda i: (i, 0)),
            pl.BlockSpec((_D, _D), lambda i: (0, 0)),
        ],
        out_specs=pl.BlockSpec((_BR, _D), lambda i: (i, 0)),
        out_shape=jax.ShapeDtypeStruct((_N, _D), jnp.float32),
    )(degc, x, w1)


def _mid_body(deg_ref, agg_ref, y1_ref, b1_ref, w2_ref, o_ref):
    dinv = lax.rsqrt(deg_ref[...] + 1.0)
    h = dinv * (agg_ref[0] + agg_ref[1] + y1_ref[...]) + b1_ref[...]
    h = jnp.maximum(h, 0.0)
    y2 = dinv * jnp.dot(h, w2_ref[...], preferred_element_type=jnp.float32)
    o_ref[...] = y2[:, :_D2]


def _mid_call(degc, agg1, y1, b1r, w2p):
    return pl.pallas_call(
        _mid_body,
        grid=(_N // _BR,),
        in_specs=[
            pl.BlockSpec((_BR, 1), lambda i: (i, 0)),
            pl.BlockSpec((_NC, _BR, _D), lambda i: (0, i, 0)),
            pl.BlockSpec((_BR, _D), lambda i: (i, 0)),
            pl.BlockSpec((1, _D), lambda i: (0, 0)),
            pl.BlockSpec((_D, _D), lambda i: (0, 0)),
        ],
        out_specs=pl.BlockSpec((_BR, _D2), lambda i: (i, 0)),
        out_shape=jax.ShapeDtypeStruct((_N, _D2), jnp.float32),
    )(degc, agg1, y1, b1r, w2p)


def _fin_body(deg_ref, agg_ref, y2_ref, b2_ref, o_ref):
    dinv = lax.rsqrt(deg_ref[...] + 1.0)
    o_ref[...] = dinv * (agg_ref[0] + agg_ref[1] + y2_ref[...]) + b2_ref[...]


def _fin_call(degc, agg2, y2, b2r):
    return pl.pallas_call(
        _fin_body,
        grid=(_N // _BR,),
        in_specs=[
            pl.BlockSpec((_BR, 1), lambda i: (i, 0)),
            pl.BlockSpec((_NC, _BR, _D2), lambda i: (0, i, 0)),
            pl.BlockSpec((_BR, _D2), lambda i: (i, 0)),
            pl.BlockSpec((1, _D2), lambda i: (0, 0)),
        ],
        out_specs=pl.BlockSpec((_BR, _D2), lambda i: (i, 0)),
        out_shape=jax.ShapeDtypeStruct((_N, _D2), jnp.float32),
    )(degc, agg2, y2, b2r)


def kernel(x, edge_index, W1, b1, W2, b2):
    x = x.astype(jnp.float32)
    npad = _EPAD - _E
    src3 = jnp.concatenate(
        [edge_index[0], jnp.zeros((npad,), jnp.int32)]).reshape(_NW, _NCH, _CH)
    dst3 = jnp.concatenate(
        [edge_index[1], jnp.full((npad,), _N, jnp.int32)]).reshape(_NW, _NCH, _CH)

    deg = _deg_call(edge_index[1])
    degc = deg[:_N].reshape(_N, 1)

    y1 = _y1_call(degc, x, W1)
    zeros1 = jnp.zeros((_NP, _D), jnp.float32)
    agg1 = _agg_call_d(y1, src3, dst3, zeros1)

    w2p = jnp.pad(W2, ((0, 0), (0, _D - W2.shape[1])))
    y2 = _mid_call(degc, agg1, y1, b1.reshape(1, _D), w2p)

    zeros2 = jnp.zeros((_NP, _D2), jnp.float32)
    agg2 = _agg_call_2(y2, src3, dst3, zeros2)

    b2p = jnp.pad(b2, (0, _D2 - b2.shape[0])).reshape(1, _D2)
    out16 = _fin_call(degc, agg2, y2, b2p)
    return out16[:, :8]


# speedup vs baseline: 2.6696x; 1.4951x over previous
import functools

import jax
import jax.numpy as jnp
from jax import lax
from jax.experimental import pallas as pl
from jax.experimental.pallas import tpu as pltpu
from jax.experimental.pallas import tpu_sc as plsc

_N = 10000
_E = 320000
_D = 128
_D2 = 16
_NP = 10240
_NC, _NS = 2, 16
_NW = _NC * _NS
_CH = 72
_NCH = 139
_EPT = _NCH * _CH
_EPAD = _NW * _EPT
_EPS = _E // _NS

_mesh = plsc.VectorSubcoreMesh(core_axis_name="c", subcore_axis_name="s")


def _deg_body(dst_hbm, deg_out, dst_v, hist_v, blk_v, acc_v, spm):
    cid = lax.axis_index("c")
    sid = lax.axis_index("s")
    zeros16 = jnp.zeros((16,), jnp.float32)
    ones16 = jnp.ones((16,), jnp.float32)

    @pl.when(cid == 0)
    def _():
        def zero(i, _):
            hist_v[pl.ds(i * 16, 16)] = zeros16
            return 0
        lax.fori_loop(0, _NP // 16, zero, 0)

        pltpu.sync_copy(dst_hbm.at[pl.ds(sid * _EPS, _EPS)], dst_v)

        def count(i, _):
            idx = dst_v[pl.ds(i * 16, 16)]
            plsc.addupdate_scatter(hist_v, [idx], ones16)
            return 0
        lax.fori_loop(0, _EPS // 16, count, 0)

        pltpu.sync_copy(hist_v, spm.at[sid])

    plsc.subcore_barrier()

    @pl.when(cid == 0)
    def _():
        pltpu.sync_copy(spm.at[:, pl.ds(sid * 640, 640)], blk_v)

        def reduce(j, _):
            s = blk_v[0, pl.ds(j * 16, 16)]
            for r in range(1, _NS):
                s = s + blk_v[r, pl.ds(j * 16, 16)]
            acc_v[pl.ds(j * 16, 16)] = s
            return 0
        lax.fori_loop(0, 640 // 16, reduce, 0)

        pltpu.sync_copy(acc_v, deg_out.at[pl.ds(sid * 640, 640)])


_sc_params = pltpu.CompilerParams(needs_layout_passes=False,
                                  use_tc_tiling_on_sc=False)

_deg_call = functools.partial(
    pl.kernel,
    out_type=jax.ShapeDtypeStruct((_NP,), jnp.float32),
    mesh=_mesh,
    compiler_params=_sc_params,
    scratch_types=[
        pltpu.VMEM((_EPS,), jnp.int32),
        pltpu.VMEM((_NP,), jnp.float32),
        pltpu.VMEM((_NS, 640), jnp.float32),
        pltpu.VMEM((640,), jnp.float32),
        pltpu.VMEM_SHARED((_NS, _NP), jnp.float32),
    ],
)(_deg_body)


def _agg_body(y_hbm, src_hbm, dst_hbm, zero_hbm, out_hbm,
              src_v, dst_v, rows_v, spm, gsem, ssem):
    cid = lax.axis_index("c")
    sid = lax.axis_index("s")
    wid = sid * _NC + cid

    pltpu.sync_copy(zero_hbm.at[pl.ds(sid * 640, 640)],
                    spm.at[pl.ds(sid * 640, 640)])
    pltpu.sync_copy(src_hbm.at[wid], src_v)
    pltpu.sync_copy(dst_hbm.at[wid], dst_v)
    plsc.subcore_barrier()

    drow = y_hbm.at[pl.ds(0, _CH)]

    pltpu.async_copy(y_hbm.at[src_v.at[0]], rows_v.at[0], gsem.at[0])
    pltpu.async_copy(y_hbm.at[src_v.at[1]], rows_v.at[1], gsem.at[1])

    def chunk(j, _):
        p = j % 3
        pltpu.make_async_copy(drow, rows_v.at[p], gsem.at[p]).wait()
        pltpu.async_copy(rows_v.at[p], spm.at[dst_v.at[j]],
                         ssem.at[p], add=True)

        @pl.when(j + 2 < _NCH)
        def _():
            q = (j + 2) % 3

            @pl.when(j >= 1)
            def _():
                pltpu.make_async_copy(drow, rows_v.at[q], ssem.at[q]).wait()

            pltpu.async_copy(y_hbm.at[src_v.at[j + 2]], rows_v.at[q],
                             gsem.at[q])
        return 0

    lax.fori_loop(0, _NCH, chunk, 0)
    for b in range(3):
        pltpu.make_async_copy(drow, rows_v.at[b], ssem.at[b]).wait()

    plsc.subcore_barrier()
    pltpu.sync_copy(spm.at[pl.ds(sid * 640, 640)],
                    out_hbm.at[cid, pl.ds(sid * 640, 640)])


def _make_agg(width):
    return functools.partial(
        pl.kernel,
        out_type=jax.ShapeDtypeStruct((_NC, _NP, width), jnp.float32),
        mesh=_mesh,
        compiler_params=_sc_params,
        scratch_types=[
            pltpu.VMEM((_NCH, _CH), jnp.int32),
            pltpu.VMEM((_NCH, _CH), jnp.int32),
            pltpu.VMEM((3, _CH, width), jnp.float32),
            pltpu.VMEM_SHARED((_NP, width), jnp.float32),
            pltpu.SemaphoreType.DMA((3,)),
            pltpu.SemaphoreType.DMA((3,)),
        ],
    )(_agg_body)


_agg_call_d = _make_agg(_D)
_agg_call_2 = _make_agg(_D2)


_BR = 1000


def _y1_body(deg_ref, x_ref, w_ref, o_ref):
    dinv = lax.rsqrt(deg_ref[...] + 1.0)
    o_ref[...] = dinv * jnp.dot(x_ref[...], w_ref[...],
                                preferred_element_type=jnp.float32)


def _y1_call(degc, x, w1):
    return pl.pallas_call(
        _y1_body,
        grid=(_N // _BR,),
        in_specs=[
            pl.BlockSpec((_BR, 1), lambda i: (i, 0)),
            pl.BlockSpec((_BR, _D), lambda i: (i, 0)),
            pl.BlockSpec((_D, _D), lambda i: (0, 0)),
        ],
        out_specs=pl.BlockSpec((_BR, _D), lambda i: (i, 0)),
        out_shape=jax.ShapeDtypeStruct((_N, _D), jnp.float32),
    )(degc, x, w1)


def _mid_body(deg_ref, agg_ref, y1_ref, b1_ref, w2_ref, o_ref):
    dinv = lax.rsqrt(deg_ref[...] + 1.0)
    h = dinv * (agg_ref[0] + agg_ref[1] + y1_ref[...]) + b1_ref[...]
    h = jnp.maximum(h, 0.0)
    y2 = dinv * jnp.dot(h, w2_ref[...], preferred_element_type=jnp.float32)
    o_ref[...] = y2[:, :_D2]


def _mid_call(degc, agg1, y1, b1r, w2p):
    return pl.pallas_call(
        _mid_body,
        grid=(_N // _BR,),
        in_specs=[
            pl.BlockSpec((_BR, 1), lambda i: (i, 0)),
            pl.BlockSpec((_NC, _BR, _D), lambda i: (0, i, 0)),
            pl.BlockSpec((_BR, _D), lambda i: (i, 0)),
            pl.BlockSpec((1, _D), lambda i: (0, 0)),
            pl.BlockSpec((_D, _D), lambda i: (0, 0)),
        ],
        out_specs=pl.BlockSpec((_BR, _D2), lambda i: (i, 0)),
        out_shape=jax.ShapeDtypeStruct((_N, _D2), jnp.float32),
    )(degc, agg1, y1, b1r, w2p)


def _fin_body(deg_ref, agg_ref, y2_ref, b2_ref, o_ref):
    dinv = lax.rsqrt(deg_ref[...] + 1.0)
    o_ref[...] = dinv * (agg_ref[0] + agg_ref[1] + y2_ref[...]) + b2_ref[...]


def _fin_call(degc, agg2, y2, b2r):
    return pl.pallas_call(
        _fin_body,
        grid=(_N // _BR,),
        in_specs=[
            pl.BlockSpec((_BR, 1), lambda i: (i, 0)),
            pl.BlockSpec((_NC, _BR, _D2), lambda i: (0, i, 0)),
            pl.BlockSpec((_BR, _D2), lambda i: (i, 0)),
            pl.BlockSpec((1, _D2), lambda i: (0, 0)),
        ],
        out_specs=pl.BlockSpec((_BR, _D2), lambda i: (i, 0)),
        out_shape=jax.ShapeDtypeStruct((_N, _D2), jnp.float32),
    )(degc, agg2, y2, b2r)


def kernel(x, edge_index, W1, b1, W2, b2):
    x = x.astype(jnp.float32)
    npad = _EPAD - _E
    src3 = jnp.concatenate(
        [edge_index[0], jnp.zeros((npad,), jnp.int32)]).reshape(_NW, _NCH, _CH)
    dst3 = jnp.concatenate(
        [edge_index[1], jnp.full((npad,), _N, jnp.int32)]).reshape(_NW, _NCH, _CH)

    deg = _deg_call(edge_index[1])
    degc = deg[:_N].reshape(_N, 1)

    y1 = _y1_call(degc, x, W1)
    zeros1 = jnp.zeros((_NP, _D), jnp.float32)
    agg1 = _agg_call_d(y1, src3, dst3, zeros1)

    w2p = jnp.pad(W2, ((0, 0), (0, _D - W2.shape[1])))
    y2 = _mid_call(degc, agg1, y1, b1.reshape(1, _D), w2p)

    zeros2 = jnp.zeros((_NP, _D2), jnp.float32)
    agg2 = _agg_call_2(y2, src3, dst3, zeros2)

    b2p = jnp.pad(b2, (0, _D2 - b2.shape[0])).reshape(1, _D2)
    out16 = _fin_call(degc, agg2, y2, b2p)
    return out16[:, :8]


# speedup vs baseline: 2.9376x; 1.1004x over previous
import functools

import jax
import jax.numpy as jnp
from jax import lax
from jax.experimental import pallas as pl
from jax.experimental.pallas import tpu as pltpu
from jax.experimental.pallas import tpu_sc as plsc

_N = 10000
_E = 320000
_D = 128
_D2 = 16
_NP = 10240
_NC, _NS = 2, 16
_NW = _NC * _NS
_CH = 72
_NCH = 139
_EPT = _NCH * _CH
_EPAD = _NW * _EPT
_EPS = _E // _NS

_mesh = plsc.VectorSubcoreMesh(core_axis_name="c", subcore_axis_name="s")


def _deg_body(dst_hbm, deg_out, dst_v, hist_v, blk_v, acc_v, spm):
    cid = lax.axis_index("c")
    sid = lax.axis_index("s")
    zeros16 = jnp.zeros((16,), jnp.float32)
    ones16 = jnp.ones((16,), jnp.float32)

    @pl.when(cid == 0)
    def _():
        def zero(i, _):
            hist_v[pl.ds(i * 16, 16)] = zeros16
            return 0
        lax.fori_loop(0, _NP // 16, zero, 0)

        pltpu.sync_copy(dst_hbm.at[pl.ds(sid * _EPS, _EPS)], dst_v)

        def count(i, _):
            idx = dst_v[pl.ds(i * 16, 16)]
            plsc.addupdate_scatter(hist_v, [idx], ones16)
            return 0
        lax.fori_loop(0, _EPS // 16, count, 0)

        pltpu.sync_copy(hist_v, spm.at[sid])

    plsc.subcore_barrier()

    @pl.when(cid == 0)
    def _():
        pltpu.sync_copy(spm.at[:, pl.ds(sid * 640, 640)], blk_v)

        def reduce(j, _):
            s = blk_v[0, pl.ds(j * 16, 16)]
            for r in range(1, _NS):
                s = s + blk_v[r, pl.ds(j * 16, 16)]
            acc_v[pl.ds(j * 16, 16)] = s
            return 0
        lax.fori_loop(0, 640 // 16, reduce, 0)

        pltpu.sync_copy(acc_v, deg_out.at[pl.ds(sid * 640, 640)])


_sc_params = pltpu.CompilerParams(needs_layout_passes=False,
                                  use_tc_tiling_on_sc=False)

_deg_call = functools.partial(
    pl.kernel,
    out_type=jax.ShapeDtypeStruct((_NP,), jnp.float32),
    mesh=_mesh,
    compiler_params=_sc_params,
    scratch_types=[
        pltpu.VMEM((_EPS,), jnp.int32),
        pltpu.VMEM((_NP,), jnp.float32),
        pltpu.VMEM((_NS, 640), jnp.float32),
        pltpu.VMEM((640,), jnp.float32),
        pltpu.VMEM_SHARED((_NS, _NP), jnp.float32),
    ],
)(_deg_body)


def _make_agg(width, nbuf):
    def _agg_body(y_hbm, src_hbm, dst_hbm, zero_hbm, out_hbm,
                  src_v, dst_v, rows_v, spm, gsem, ssem):
        cid = lax.axis_index("c")
        sid = lax.axis_index("s")
        wid = sid * _NC + cid

        pltpu.sync_copy(zero_hbm.at[pl.ds(sid * 640, 640)],
                        spm.at[pl.ds(sid * 640, 640)])
        pltpu.sync_copy(src_hbm.at[wid], src_v)
        pltpu.sync_copy(dst_hbm.at[wid], dst_v)
        plsc.subcore_barrier()

        drow = y_hbm.at[pl.ds(0, _CH)]

        for b in range(nbuf - 1):
            pltpu.async_copy(y_hbm.at[src_v.at[b]], rows_v.at[b], gsem.at[b])

        def chunk(j, _):
            p = j % nbuf
            pltpu.make_async_copy(drow, rows_v.at[p], gsem.at[p]).wait()
            pltpu.async_copy(rows_v.at[p], spm.at[dst_v.at[j]],
                             ssem.at[p], add=True)

            @pl.when(j + nbuf - 1 < _NCH)
            def _():
                q = (j + nbuf - 1) % nbuf

                @pl.when(j >= 1)
                def _():
                    pltpu.make_async_copy(drow, rows_v.at[q],
                                          ssem.at[q]).wait()

                pltpu.async_copy(y_hbm.at[src_v.at[j + nbuf - 1]],
                                 rows_v.at[q], gsem.at[q])
            return 0

        lax.fori_loop(0, _NCH, chunk, 0)
        for b in range(nbuf):
            pltpu.make_async_copy(drow, rows_v.at[b], ssem.at[b]).wait()

        plsc.subcore_barrier()
        pltpu.sync_copy(spm.at[pl.ds(sid * 640, 640)],
                        out_hbm.at[cid, pl.ds(sid * 640, 640)])

    return functools.partial(
        pl.kernel,
        out_type=jax.ShapeDtypeStruct((_NC, _NP, width), jnp.float32),
        mesh=_mesh,
        compiler_params=_sc_params,
        scratch_types=[
            pltpu.VMEM((_NCH, _CH), jnp.int32),
            pltpu.VMEM((_NCH, _CH), jnp.int32),
            pltpu.VMEM((nbuf, _CH, width), jnp.float32),
            pltpu.VMEM_SHARED((_NP, width), jnp.float32),
            pltpu.SemaphoreType.DMA((nbuf,)),
            pltpu.SemaphoreType.DMA((nbuf,)),
        ],
    )(_agg_body)


_agg_call_d = _make_agg(_D, 3)
_agg_call_2 = _make_agg(_D2, 6)


_BR = 1000


def _y1_body(deg_ref, x_ref, w_ref, o_ref):
    dinv = lax.rsqrt(deg_ref[...] + 1.0)
    o_ref[...] = dinv * jnp.dot(x_ref[...], w_ref[...],
                                preferred_element_type=jnp.float32)


def _y1_call(degc, x, w1):
    return pl.pallas_call(
        _y1_body,
        grid=(_N // _BR,),
        in_specs=[
            pl.BlockSpec((_BR, 1), lambda i: (i, 0)),
            pl.BlockSpec((_BR, _D), lambda i: (i, 0)),
            pl.BlockSpec((_D, _D), lambda i: (0, 0)),
        ],
        out_specs=pl.BlockSpec((_BR, _D), lambda i: (i, 0)),
        out_shape=jax.ShapeDtypeStruct((_N, _D), jnp.float32),
    )(degc, x, w1)


def _mid_body(deg_ref, agg_ref, y1_ref, b1_ref, w2_ref, o_ref):
    dinv = lax.rsqrt(deg_ref[...] + 1.0)
    h = dinv * (agg_ref[0] + agg_ref[1] + y1_ref[...]) + b1_ref[...]
    h = jnp.maximum(h, 0.0)
    y2 = dinv * jnp.dot(h, w2_ref[...], preferred_element_type=jnp.float32)
    o_ref[...] = y2[:, :_D2]


def _mid_call(degc, agg1, y1, b1r, w2p):
    return pl.pallas_call(
        _mid_body,
        grid=(_N // _BR,),
        in_specs=[
            pl.BlockSpec((_BR, 1), lambda i: (i, 0)),
            pl.BlockSpec((_NC, _BR, _D), lambda i: (0, i, 0)),
            pl.BlockSpec((_BR, _D), lambda i: (i, 0)),
            pl.BlockSpec((1, _D), lambda i: (0, 0)),
            pl.BlockSpec((_D, _D), lambda i: (0, 0)),
        ],
        out_specs=pl.BlockSpec((_BR, _D2), lambda i: (i, 0)),
        out_shape=jax.ShapeDtypeStruct((_N, _D2), jnp.float32),
    )(degc, agg1, y1, b1r, w2p)


def _fin_body(deg_ref, agg_ref, y2_ref, b2_ref, o_ref):
    dinv = lax.rsqrt(deg_ref[...] + 1.0)
    o_ref[...] = dinv * (agg_ref[0] + agg_ref[1] + y2_ref[...]) + b2_ref[...]


def _fin_call(degc, agg2, y2, b2r):
    return pl.pallas_call(
        _fin_body,
        grid=(_N // _BR,),
        in_specs=[
            pl.BlockSpec((_BR, 1), lambda i: (i, 0)),
            pl.BlockSpec((_NC, _BR, _D2), lambda i: (0, i, 0)),
            pl.BlockSpec((_BR, _D2), lambda i: (i, 0)),
            pl.BlockSpec((1, _D2), lambda i: (0, 0)),
        ],
        out_specs=pl.BlockSpec((_BR, _D2), lambda i: (i, 0)),
        out_shape=jax.ShapeDtypeStruct((_N, _D2), jnp.float32),
    )(degc, agg2, y2, b2r)


def kernel(x, edge_index, W1, b1, W2, b2):
    x = x.astype(jnp.float32)
    npad = _EPAD - _E
    src3 = jnp.concatenate(
        [edge_index[0], jnp.zeros((npad,), jnp.int32)]).reshape(_NW, _NCH, _CH)
    dst3 = jnp.concatenate(
        [edge_index[1], jnp.full((npad,), _N, jnp.int32)]).reshape(_NW, _NCH, _CH)

    deg = _deg_call(edge_index[1])
    degc = deg[:_N].reshape(_N, 1)

    y1 = _y1_call(degc, x, W1)
    zeros1 = jnp.zeros((_NP, _D), jnp.float32)
    agg1 = _agg_call_d(y1, src3, dst3, zeros1)

    w2p = jnp.pad(W2, ((0, 0), (0, _D - W2.shape[1])))
    y2 = _mid_call(degc, agg1, y1, b1.reshape(1, _D), w2p)

    zeros2 = jnp.zeros((_NP, _D2), jnp.float32)
    agg2 = _agg_call_2(y2, src3, dst3, zeros2)

    b2p = jnp.pad(b2, (0, _D2 - b2.shape[0])).reshape(1, _D2)
    out16 = _fin_call(degc, agg2, y2, b2p)
    return out16[:, :8]


# speedup vs baseline: 3.0600x; 1.0417x over previous
import functools

import jax
import jax.numpy as jnp
from jax import lax
from jax.experimental import pallas as pl
from jax.experimental.pallas import tpu as pltpu
from jax.experimental.pallas import tpu_sc as plsc

_N = 10000
_E = 320000
_D = 128
_D2 = 16
_NP = 10240
_NC, _NS = 2, 16
_NW = _NC * _NS
_CH = 72
_NCH = 139
_EPT = _NCH * _CH
_EPAD = _NW * _EPT
_EPS = _E // _NS

_mesh = plsc.VectorSubcoreMesh(core_axis_name="c", subcore_axis_name="s")


def _deg_body(dst_hbm, deg_out, dst_v, hist_v, blk_v, acc_v, spm):
    cid = lax.axis_index("c")
    sid = lax.axis_index("s")
    zeros16 = jnp.zeros((16,), jnp.float32)
    ones16 = jnp.ones((16,), jnp.float32)

    @pl.when(cid == 0)
    def _():
        def zero(i, _):
            hist_v[pl.ds(i * 16, 16)] = zeros16
            return 0
        lax.fori_loop(0, _NP // 16, zero, 0)

        pltpu.sync_copy(dst_hbm.at[pl.ds(sid * _EPS, _EPS)], dst_v)

        def count(i, _):
            idx = dst_v[pl.ds(i * 16, 16)]
            plsc.addupdate_scatter(hist_v, [idx], ones16)
            return 0
        lax.fori_loop(0, _EPS // 16, count, 0)

        pltpu.sync_copy(hist_v, spm.at[sid])

    plsc.subcore_barrier()

    @pl.when(cid == 0)
    def _():
        pltpu.sync_copy(spm.at[:, pl.ds(sid * 640, 640)], blk_v)

        def reduce(j, _):
            s = blk_v[0, pl.ds(j * 16, 16)]
            for r in range(1, _NS):
                s = s + blk_v[r, pl.ds(j * 16, 16)]
            acc_v[pl.ds(j * 16, 16)] = s
            return 0
        lax.fori_loop(0, 640 // 16, reduce, 0)

        pltpu.sync_copy(acc_v, deg_out.at[pl.ds(sid * 640, 640)])


_sc_params = pltpu.CompilerParams(needs_layout_passes=False,
                                  use_tc_tiling_on_sc=False)

_deg_call = functools.partial(
    pl.kernel,
    out_type=jax.ShapeDtypeStruct((_NP,), jnp.float32),
    mesh=_mesh,
    compiler_params=_sc_params,
    scratch_types=[
        pltpu.VMEM((_EPS,), jnp.int32),
        pltpu.VMEM((_NP,), jnp.float32),
        pltpu.VMEM((_NS, 640), jnp.float32),
        pltpu.VMEM((640,), jnp.float32),
        pltpu.VMEM_SHARED((_NS, _NP), jnp.float32),
    ],
)(_deg_body)


def _make_agg(width, nbuf):
    def _agg_body(y_hbm, src_hbm, dst_hbm, zero_hbm, out_hbm,
                  src_v, dst_v, rows_v, spm, gsem, ssem):
        cid = lax.axis_index("c")
        sid = lax.axis_index("s")
        wid = sid * _NC + cid

        pltpu.sync_copy(zero_hbm.at[pl.ds(sid * 640, 640)],
                        spm.at[pl.ds(sid * 640, 640)])
        pltpu.sync_copy(src_hbm.at[wid], src_v)
        pltpu.sync_copy(dst_hbm.at[wid], dst_v)
        plsc.subcore_barrier()

        drow = y_hbm.at[pl.ds(0, _CH)]

        for b in range(nbuf - 1):
            pltpu.async_copy(y_hbm.at[src_v.at[b]], rows_v.at[b], gsem.at[b])

        def chunk(j, _):
            p = j % nbuf
            pltpu.make_async_copy(drow, rows_v.at[p], gsem.at[p]).wait()
            pltpu.async_copy(rows_v.at[p], spm.at[dst_v.at[j]],
                             ssem.at[p], add=True)

            @pl.when(j + nbuf - 1 < _NCH)
            def _():
                q = (j + nbuf - 1) % nbuf

                @pl.when(j >= 1)
                def _():
                    pltpu.make_async_copy(drow, rows_v.at[q],
                                          ssem.at[q]).wait()

                pltpu.async_copy(y_hbm.at[src_v.at[j + nbuf - 1]],
                                 rows_v.at[q], gsem.at[q])
            return 0

        lax.fori_loop(0, _NCH, chunk, 0)
        for b in range(nbuf):
            pltpu.make_async_copy(drow, rows_v.at[b], ssem.at[b]).wait()

        plsc.subcore_barrier()
        pltpu.sync_copy(spm.at[pl.ds(sid * 640, 640)],
                        out_hbm.at[cid, pl.ds(sid * 640, 640)])

    return functools.partial(
        pl.kernel,
        out_type=jax.ShapeDtypeStruct((_NC, _NP, width), jnp.float32),
        mesh=_mesh,
        compiler_params=_sc_params,
        scratch_types=[
            pltpu.VMEM((_NCH, _CH), jnp.int32),
            pltpu.VMEM((_NCH, _CH), jnp.int32),
            pltpu.VMEM((nbuf, _CH, width), jnp.float32),
            pltpu.VMEM_SHARED((_NP, width), jnp.float32),
            pltpu.SemaphoreType.DMA((nbuf,)),
            pltpu.SemaphoreType.DMA((nbuf,)),
        ],
    )(_agg_body)


_agg_call_d = _make_agg(_D, 3)

_NB2 = 8


def _agg2_body(y_hbm, src_hbm, dst_hbm, zero_hbm, out_hbm,
               src_v, dst_v, rows_v, spm_y, spm, gsem, ssem):
    cid = lax.axis_index("c")
    sid = lax.axis_index("s")
    wid = sid * _NC + cid

    pltpu.sync_copy(zero_hbm.at[pl.ds(sid * 640, 640)],
                    spm.at[pl.ds(sid * 640, 640)])
    pltpu.sync_copy(y_hbm.at[pl.ds(sid * 625, 625)],
                    spm_y.at[pl.ds(sid * 625, 625)])
    pltpu.sync_copy(src_hbm.at[wid], src_v)
    pltpu.sync_copy(dst_hbm.at[wid], dst_v)
    plsc.subcore_barrier()

    drow = spm_y.at[pl.ds(0, _CH)]

    for b in range(_NB2 - 1):
        pltpu.async_copy(spm_y.at[src_v.at[b]], rows_v.at[b], gsem.at[b])

    def chunk(j, _):
        p = j % _NB2
        pltpu.make_async_copy(drow, rows_v.at[p], gsem.at[p]).wait()
        pltpu.async_copy(rows_v.at[p], spm.at[dst_v.at[j]],
                         ssem.at[p], add=True)

        @pl.when(j + _NB2 - 1 < _NCH)
        def _():
            q = (j + _NB2 - 1) % _NB2

            @pl.when(j >= 1)
            def _():
                pltpu.make_async_copy(drow, rows_v.at[q], ssem.at[q]).wait()

            pltpu.async_copy(spm_y.at[src_v.at[j + _NB2 - 1]],
                             rows_v.at[q], gsem.at[q])
        return 0

    lax.fori_loop(0, _NCH, chunk, 0)
    for b in range(_NB2):
        pltpu.make_async_copy(drow, rows_v.at[b], ssem.at[b]).wait()

    plsc.subcore_barrier()
    pltpu.sync_copy(spm.at[pl.ds(sid * 640, 640)],
                    out_hbm.at[cid, pl.ds(sid * 640, 640)])


_agg_call_2 = functools.partial(
    pl.kernel,
    out_type=jax.ShapeDtypeStruct((_NC, _NP, _D2), jnp.float32),
    mesh=_mesh,
    compiler_params=_sc_params,
    scratch_types=[
        pltpu.VMEM((_NCH, _CH), jnp.int32),
        pltpu.VMEM((_NCH, _CH), jnp.int32),
        pltpu.VMEM((_NB2, _CH, _D2), jnp.float32),
        pltpu.VMEM_SHARED((_N, _D2), jnp.float32),
        pltpu.VMEM_SHARED((_NP, _D2), jnp.float32),
        pltpu.SemaphoreType.DMA((_NB2,)),
        pltpu.SemaphoreType.DMA((_NB2,)),
    ],
)(_agg2_body)


_BR = 1000


def _y1_body(deg_ref, x_ref, w_ref, o_ref):
    dinv = lax.rsqrt(deg_ref[...] + 1.0)
    o_ref[...] = dinv * jnp.dot(x_ref[...], w_ref[...],
                                preferred_element_type=jnp.float32)


def _y1_call(degc, x, w1):
    return pl.pallas_call(
        _y1_body,
        grid=(_N // _BR,),
        in_specs=[
            pl.BlockSpec((_BR, 1), lambda i: (i, 0)),
            pl.BlockSpec((_BR, _D), lambda i: (i, 0)),
            pl.BlockSpec((_D, _D), lambda i: (0, 0)),
        ],
        out_specs=pl.BlockSpec((_BR, _D), lambda i: (i, 0)),
        out_shape=jax.ShapeDtypeStruct((_N, _D), jnp.float32),
    )(degc, x, w1)


def _mid_body(deg_ref, agg_ref, y1_ref, b1_ref, w2_ref, o_ref):
    dinv = lax.rsqrt(deg_ref[...] + 1.0)
    h = dinv * (agg_ref[0] + agg_ref[1] + y1_ref[...]) + b1_ref[...]
    h = jnp.maximum(h, 0.0)
    y2 = dinv * jnp.dot(h, w2_ref[...], preferred_element_type=jnp.float32)
    o_ref[...] = y2[:, :_D2]


def _mid_call(degc, agg1, y1, b1r, w2p):
    return pl.pallas_call(
        _mid_body,
        grid=(_N // _BR,),
        in_specs=[
            pl.BlockSpec((_BR, 1), lambda i: (i, 0)),
            pl.BlockSpec((_NC, _BR, _D), lambda i: (0, i, 0)),
            pl.BlockSpec((_BR, _D), lambda i: (i, 0)),
            pl.BlockSpec((1, _D), lambda i: (0, 0)),
            pl.BlockSpec((_D, _D), lambda i: (0, 0)),
        ],
        out_specs=pl.BlockSpec((_BR, _D2), lambda i: (i, 0)),
        out_shape=jax.ShapeDtypeStruct((_N, _D2), jnp.float32),
    )(degc, agg1, y1, b1r, w2p)


def _fin_body(deg_ref, agg_ref, y2_ref, b2_ref, o_ref):
    dinv = lax.rsqrt(deg_ref[...] + 1.0)
    o_ref[...] = dinv * (agg_ref[0] + agg_ref[1] + y2_ref[...]) + b2_ref[...]


def _fin_call(degc, agg2, y2, b2r):
    return pl.pallas_call(
        _fin_body,
        grid=(_N // _BR,),
        in_specs=[
            pl.BlockSpec((_BR, 1), lambda i: (i, 0)),
            pl.BlockSpec((_NC, _BR, _D2), lambda i: (0, i, 0)),
            pl.BlockSpec((_BR, _D2), lambda i: (i, 0)),
            pl.BlockSpec((1, _D2), lambda i: (0, 0)),
        ],
        out_specs=pl.BlockSpec((_BR, _D2), lambda i: (i, 0)),
        out_shape=jax.ShapeDtypeStruct((_N, _D2), jnp.float32),
    )(degc, agg2, y2, b2r)


def kernel(x, edge_index, W1, b1, W2, b2):
    x = x.astype(jnp.float32)
    npad = _EPAD - _E
    src3 = jnp.concatenate(
        [edge_index[0], jnp.zeros((npad,), jnp.int32)]).reshape(_NW, _NCH, _CH)
    dst3 = jnp.concatenate(
        [edge_index[1], jnp.full((npad,), _N, jnp.int32)]).reshape(_NW, _NCH, _CH)

    deg = _deg_call(edge_index[1])
    degc = deg[:_N].reshape(_N, 1)

    y1 = _y1_call(degc, x, W1)
    zeros1 = jnp.zeros((_NP, _D), jnp.float32)
    agg1 = _agg_call_d(y1, src3, dst3, zeros1)

    w2p = jnp.pad(W2, ((0, 0), (0, _D - W2.shape[1])))
    y2 = _mid_call(degc, agg1, y1, b1.reshape(1, _D), w2p)

    zeros2 = jnp.zeros((_NP, _D2), jnp.float32)
    agg2 = _agg_call_2(y2, src3, dst3, zeros2)

    b2p = jnp.pad(b2, (0, _D2 - b2.shape[0])).reshape(1, _D2)
    out16 = _fin_call(degc, agg2, y2, b2p)
    return out16[:, :8]


# speedup vs baseline: 3.0702x; 1.0033x over previous
import functools

import jax
import jax.numpy as jnp
from jax import lax
from jax.experimental import pallas as pl
from jax.experimental.pallas import tpu as pltpu
from jax.experimental.pallas import tpu_sc as plsc

_N = 10000
_E = 320000
_D = 128
_D2 = 16
_NP = 10240
_NC, _NS = 2, 16
_NW = _NC * _NS
_CH = 72
_NCH = 139
_EPT = _NCH * _CH
_EPAD = _NW * _EPT
_EPW = _E // _NW

_mesh = plsc.VectorSubcoreMesh(core_axis_name="c", subcore_axis_name="s")


def _deg_body(dst_hbm, deg_out, dst_v, hist_v, blk_v, acc_v, spm):
    cid = lax.axis_index("c")
    sid = lax.axis_index("s")
    wid = sid * _NC + cid
    zeros16 = jnp.zeros((16,), jnp.float32)
    ones16 = jnp.ones((16,), jnp.float32)

    def zero(i, _):
        hist_v[pl.ds(i * 16, 16)] = zeros16
        return 0
    lax.fori_loop(0, _NP // 16, zero, 0)

    pltpu.sync_copy(dst_hbm.at[pl.ds(wid * _EPW, _EPW)], dst_v)

    def count(i, _):
        idx = dst_v[pl.ds(i * 16, 16)]
        plsc.addupdate_scatter(hist_v, [idx], ones16)
        return 0
    lax.fori_loop(0, _EPW // 16, count, 0)

    pltpu.sync_copy(hist_v, spm.at[sid])

    plsc.subcore_barrier()

    pltpu.sync_copy(spm.at[:, pl.ds(sid * 640, 640)], blk_v)

    def reduce(j, _):
        s = blk_v[0, pl.ds(j * 16, 16)]
        for r in range(1, _NS):
            s = s + blk_v[r, pl.ds(j * 16, 16)]
        acc_v[pl.ds(j * 16, 16)] = s
        return 0
    lax.fori_loop(0, 640 // 16, reduce, 0)

    pltpu.sync_copy(acc_v, deg_out.at[cid, pl.ds(sid * 640, 640)])


_sc_params = pltpu.CompilerParams(needs_layout_passes=False,
                                  use_tc_tiling_on_sc=False)

_deg_call = functools.partial(
    pl.kernel,
    out_type=jax.ShapeDtypeStruct((_NC, _NP), jnp.float32),
    mesh=_mesh,
    compiler_params=_sc_params,
    scratch_types=[
        pltpu.VMEM((_EPW,), jnp.int32),
        pltpu.VMEM((_NP,), jnp.float32),
        pltpu.VMEM((_NS, 640), jnp.float32),
        pltpu.VMEM((640,), jnp.float32),
        pltpu.VMEM_SHARED((_NS, _NP), jnp.float32),
    ],
)(_deg_body)


def _make_agg(width, nbuf):
    def _agg_body(y_hbm, src_hbm, dst_hbm, zero_hbm, out_hbm,
                  src_v, dst_v, rows_v, spm, gsem, ssem):
        cid = lax.axis_index("c")
        sid = lax.axis_index("s")
        wid = sid * _NC + cid

        pltpu.sync_copy(zero_hbm.at[pl.ds(sid * 640, 640)],
                        spm.at[pl.ds(sid * 640, 640)])
        pltpu.sync_copy(src_hbm.at[wid], src_v)
        pltpu.sync_copy(dst_hbm.at[wid], dst_v)
        plsc.subcore_barrier()

        drow = y_hbm.at[pl.ds(0, _CH)]

        for b in range(nbuf - 1):
            pltpu.async_copy(y_hbm.at[src_v.at[b]], rows_v.at[b], gsem.at[b])

        def chunk(j, _):
            p = j % nbuf
            pltpu.make_async_copy(drow, rows_v.at[p], gsem.at[p]).wait()
            pltpu.async_copy(rows_v.at[p], spm.at[dst_v.at[j]],
                             ssem.at[p], add=True)

            @pl.when(j + nbuf - 1 < _NCH)
            def _():
                q = (j + nbuf - 1) % nbuf

                @pl.when(j >= 1)
                def _():
                    pltpu.make_async_copy(drow, rows_v.at[q],
                                          ssem.at[q]).wait()

                pltpu.async_copy(y_hbm.at[src_v.at[j + nbuf - 1]],
                                 rows_v.at[q], gsem.at[q])
            return 0

        lax.fori_loop(0, _NCH, chunk, 0)
        for b in range(nbuf):
            pltpu.make_async_copy(drow, rows_v.at[b], ssem.at[b]).wait()

        plsc.subcore_barrier()
        pltpu.sync_copy(spm.at[pl.ds(sid * 640, 640)],
                        out_hbm.at[cid, pl.ds(sid * 640, 640)])

    return functools.partial(
        pl.kernel,
        out_type=jax.ShapeDtypeStruct((_NC, _NP, width), jnp.float32),
        mesh=_mesh,
        compiler_params=_sc_params,
        scratch_types=[
            pltpu.VMEM((_NCH, _CH), jnp.int32),
            pltpu.VMEM((_NCH, _CH), jnp.int32),
            pltpu.VMEM((nbuf, _CH, width), jnp.float32),
            pltpu.VMEM_SHARED((_NP, width), jnp.float32),
            pltpu.SemaphoreType.DMA((nbuf,)),
            pltpu.SemaphoreType.DMA((nbuf,)),
        ],
    )(_agg_body)


_agg_call_d = _make_agg(_D, 3)

_NB2 = 8


def _agg2_body(y_hbm, src_hbm, dst_hbm, zero_hbm, out_hbm,
               src_v, dst_v, rows_v, spm_y, spm, gsem, ssem):
    cid = lax.axis_index("c")
    sid = lax.axis_index("s")
    wid = sid * _NC + cid

    pltpu.sync_copy(zero_hbm.at[pl.ds(sid * 640, 640)],
                    spm.at[pl.ds(sid * 640, 640)])
    pltpu.sync_copy(y_hbm.at[pl.ds(sid * 625, 625)],
                    spm_y.at[pl.ds(sid * 625, 625)])
    pltpu.sync_copy(src_hbm.at[wid], src_v)
    pltpu.sync_copy(dst_hbm.at[wid], dst_v)
    plsc.subcore_barrier()

    drow = spm_y.at[pl.ds(0, _CH)]

    for b in range(_NB2 - 1):
        pltpu.async_copy(spm_y.at[src_v.at[b]], rows_v.at[b], gsem.at[b])

    def chunk(j, _):
        p = j % _NB2
        pltpu.make_async_copy(drow, rows_v.at[p], gsem.at[p]).wait()
        pltpu.async_copy(rows_v.at[p], spm.at[dst_v.at[j]],
                         ssem.at[p], add=True)

        @pl.when(j + _NB2 - 1 < _NCH)
        def _():
            q = (j + _NB2 - 1) % _NB2

            @pl.when(j >= 1)
            def _():
                pltpu.make_async_copy(drow, rows_v.at[q], ssem.at[q]).wait()

            pltpu.async_copy(spm_y.at[src_v.at[j + _NB2 - 1]],
                             rows_v.at[q], gsem.at[q])
        return 0

    lax.fori_loop(0, _NCH, chunk, 0)
    for b in range(_NB2):
        pltpu.make_async_copy(drow, rows_v.at[b], ssem.at[b]).wait()

    plsc.subcore_barrier()
    pltpu.sync_copy(spm.at[pl.ds(sid * 640, 640)],
                    out_hbm.at[cid, pl.ds(sid * 640, 640)])


_agg_call_2 = functools.partial(
    pl.kernel,
    out_type=jax.ShapeDtypeStruct((_NC, _NP, _D2), jnp.float32),
    mesh=_mesh,
    compiler_params=_sc_params,
    scratch_types=[
        pltpu.VMEM((_NCH, _CH), jnp.int32),
        pltpu.VMEM((_NCH, _CH), jnp.int32),
        pltpu.VMEM((_NB2, _CH, _D2), jnp.float32),
        pltpu.VMEM_SHARED((_N, _D2), jnp.float32),
        pltpu.VMEM_SHARED((_NP, _D2), jnp.float32),
        pltpu.SemaphoreType.DMA((_NB2,)),
        pltpu.SemaphoreType.DMA((_NB2,)),
    ],
)(_agg2_body)


_BR = 1000


def _y1_body(deg_ref, x_ref, w_ref, o_ref, d_ref):
    dsum = deg_ref[0] + deg_ref[1]
    dinv = lax.rsqrt(dsum + 1.0)
    o_ref[...] = dinv * jnp.dot(x_ref[...], w_ref[...],
                                preferred_element_type=jnp.float32)
    d_ref[...] = dsum


def _y1_call(degp, x, w1):
    return pl.pallas_call(
        _y1_body,
        grid=(_N // _BR,),
        in_specs=[
            pl.BlockSpec((_NC, _BR, 1), lambda i: (0, i, 0)),
            pl.BlockSpec((_BR, _D), lambda i: (i, 0)),
            pl.BlockSpec((_D, _D), lambda i: (0, 0)),
        ],
        out_specs=[
            pl.BlockSpec((_BR, _D), lambda i: (i, 0)),
            pl.BlockSpec((_BR, 1), lambda i: (i, 0)),
        ],
        out_shape=[
            jax.ShapeDtypeStruct((_N, _D), jnp.float32),
            jax.ShapeDtypeStruct((_N, 1), jnp.float32),
        ],
    )(degp, x, w1)


def _mid_body(deg_ref, agg_ref, y1_ref, b1_ref, w2_ref, o_ref):
    dinv = lax.rsqrt(deg_ref[...] + 1.0)
    h = dinv * (agg_ref[0] + agg_ref[1] + y1_ref[...]) + b1_ref[...]
    h = jnp.maximum(h, 0.0)
    y2 = dinv * jnp.dot(h, w2_ref[...], preferred_element_type=jnp.float32)
    o_ref[...] = y2[:, :_D2]


def _mid_call(degc, agg1, y1, b1r, w2p):
    return pl.pallas_call(
        _mid_body,
        grid=(_N // _BR,),
        in_specs=[
            pl.BlockSpec((_BR, 1), lambda i: (i, 0)),
            pl.BlockSpec((_NC, _BR, _D), lambda i: (0, i, 0)),
            pl.BlockSpec((_BR, _D), lambda i: (i, 0)),
            pl.BlockSpec((1, _D), lambda i: (0, 0)),
            pl.BlockSpec((_D, _D), lambda i: (0, 0)),
        ],
        out_specs=pl.BlockSpec((_BR, _D2), lambda i: (i, 0)),
        out_shape=jax.ShapeDtypeStruct((_N, _D2), jnp.float32),
    )(degc, agg1, y1, b1r, w2p)


def _fin_body(deg_ref, agg_ref, y2_ref, b2_ref, o_ref):
    dinv = lax.rsqrt(deg_ref[...] + 1.0)
    o_ref[...] = dinv * (agg_ref[0] + agg_ref[1] + y2_ref[...]) + b2_ref[...]


def _fin_call(degc, agg2, y2, b2r):
    return pl.pallas_call(
        _fin_body,
        grid=(_N // _BR,),
        in_specs=[
            pl.BlockSpec((_BR, 1), lambda i: (i, 0)),
            pl.BlockSpec((_NC, _BR, _D2), lambda i: (0, i, 0)),
            pl.BlockSpec((_BR, _D2), lambda i: (i, 0)),
            pl.BlockSpec((1, _D2), lambda i: (0, 0)),
        ],
        out_specs=pl.BlockSpec((_BR, _D2), lambda i: (i, 0)),
        out_shape=jax.ShapeDtypeStruct((_N, _D2), jnp.float32),
    )(degc, agg2, y2, b2r)


def kernel(x, edge_index, W1, b1, W2, b2):
    x = x.astype(jnp.float32)
    npad = _EPAD - _E
    src3 = jnp.concatenate(
        [edge_index[0], jnp.zeros((npad,), jnp.int32)]).reshape(_NW, _NCH, _CH)
    dst3 = jnp.concatenate(
        [edge_index[1], jnp.full((npad,), _N, jnp.int32)]).reshape(_NW, _NCH, _CH)

    deg = _deg_call(edge_index[1])
    degp = deg[:, :_N].reshape(_NC, _N, 1)

    y1, degc = _y1_call(degp, x, W1)
    zeros1 = jnp.zeros((_NP, _D), jnp.float32)
    agg1 = _agg_call_d(y1, src3, dst3, zeros1)

    w2p = jnp.pad(W2, ((0, 0), (0, _D - W2.shape[1])))
    y2 = _mid_call(degc, agg1, y1, b1.reshape(1, _D), w2p)

    zeros2 = jnp.zeros((_NP, _D2), jnp.float32)
    agg2 = _agg_call_2(y2, src3, dst3, zeros2)

    b2p = jnp.pad(b2, (0, _D2 - b2.shape[0])).reshape(1, _D2)
    out16 = _fin_call(degc, agg2, y2, b2p)
    return out16[:, :8]
